# Initial kernel scaffold; baseline (speedup 1.0000x reference)
import jax
import jax.numpy as jnp
from jax.experimental import pallas as pl


def kernel(observations, norm_factors):
    raise NotImplementedError("write your pallas kernel here")


# speedup vs baseline: 1.5907x; 1.5907x over previous
import functools

import jax
import jax.numpy as jnp
from jax import lax
from jax.experimental import pallas as pl
from jax.experimental.pallas import tpu as pltpu
from jax.experimental.pallas import tpu_sc as plsc

B = 4096
T = 200
NTOK = B * T
NWORD = NTOK * 3
NC = 2
NS = 16
NW = NC * NS
TOK_PER_W = NTOK // NW
CHUNK = 6400
NCHUNK = TOK_PER_W // CHUNK
CW = CHUNK * 3
L = 16


def kernel(observations, norm_factors):
    obs_flat = observations.reshape(-1)

    mesh = plsc.VectorSubcoreMesh(core_axis_name="c", subcore_axis_name="s")

    @functools.partial(
        pl.kernel,
        mesh=mesh,
        out_type=jax.ShapeDtypeStruct((NWORD,), jnp.float32),
        compiler_params=pltpu.CompilerParams(needs_layout_passes=False),
        scratch_types=[
            pltpu.VMEM((256,), jnp.float32),
            pltpu.VMEM((256,), jnp.float32),
            pltpu.VMEM((CW,), jnp.int32),
            pltpu.VMEM((CW,), jnp.float32),
        ],
    )
    def sc_kernel(obs_hbm, nf_hbm, out_hbm, nf_v, rcp_v, in_v, out_v):
        wid = lax.axis_index("s") * NC + lax.axis_index("c")

        pltpu.sync_copy(nf_hbm, nf_v)
        for i in range(256 // L):
            rcp_v[pl.ds(i * L, L)] = 1.0 / nf_v[pl.ds(i * L, L)]

        lane = lax.iota(jnp.int32, L)
        pos1_c = lane * 3 + 1
        pos2_c = lane * 3 + 2

        base_w = wid * (TOK_PER_W * 3)

        for c in range(NCHUNK):
            off = base_w + c * CW
            pltpu.sync_copy(obs_hbm.at[pl.ds(off, CW)], in_v)

            def cast_body(i, carry):
                w = i * (4 * L)
                for u in range(4):
                    out_v[pl.ds(w + u * L, L)] = (
                        in_v[pl.ds(w + u * L, L)].astype(jnp.float32))
                return carry

            lax.fori_loop(0, CW // (4 * L), cast_body, 0)

            def fix_body(j, carry):
                b = j * (4 * 3 * L)
                for u in range(4):
                    p1 = pos1_c + (b + u * 3 * L)
                    p2 = pos2_c + (b + u * 3 * L)
                    idx = plsc.load_gather(in_v, [p1])
                    rcp = plsc.load_gather(rcp_v, [idx])
                    val = plsc.load_gather(out_v, [p2])
                    plsc.store_scatter(out_v, [p2], val * rcp)
                return carry

            lax.fori_loop(0, CHUNK // (4 * L), fix_body, 0)

            pltpu.sync_copy(out_v, out_hbm.at[pl.ds(off, CW)])

    out_flat = sc_kernel(obs_flat, norm_factors)
    return out_flat.reshape(B, T, 3)


# speedup vs baseline: 88.5896x; 55.6917x over previous
import functools

import jax
import jax.numpy as jnp
from jax import lax
from jax.experimental import pallas as pl
from jax.experimental.pallas import tpu as pltpu
from jax.experimental.pallas import tpu_sc as plsc

B = 4096
T = 200
NPLANE = 3
ROWS = T
COLS = B
TROW = 8
NTROW = ROWS // TROW
HALF = COLS // 2
L = 16


def kernel(observations, norm_factors):
    xt = jnp.transpose(observations, (2, 1, 0))

    mesh = plsc.VectorSubcoreMesh(core_axis_name="c", subcore_axis_name="s")

    @functools.partial(
        pl.kernel,
        mesh=mesh,
        out_type=jax.ShapeDtypeStruct((NPLANE, ROWS, COLS), jnp.float32),
        compiler_params=pltpu.CompilerParams(needs_layout_passes=False),
        scratch_types=[
            pltpu.VMEM((256,), jnp.float32),
            pltpu.VMEM((256,), jnp.float32),
            pltpu.VMEM((TROW, HALF), jnp.int32),
            pltpu.VMEM((TROW, HALF), jnp.int32),
            pltpu.VMEM((TROW, HALF), jnp.float32),
            pltpu.VMEM((TROW, HALF), jnp.float32),
        ],
    )
    def sc_kernel(x_hbm, nf_hbm, out_hbm, nf_v, rcp_v, i1v, i2v, o1v, o2v):
        w = lax.axis_index("s") * 2 + lax.axis_index("c")

        pltpu.sync_copy(nf_hbm, nf_v)
        for i in range(256 // L):
            rcp_v[pl.ds(i * L, L)] = 1.0 / nf_v[pl.ds(i * L, L)]

        def joint_half(row, h):
            rs = pl.ds(row * TROW, TROW)
            cs = pl.ds(h * HALF, HALF)
            pltpu.sync_copy(x_hbm.at[1, rs, cs], i1v)
            pltpu.sync_copy(x_hbm.at[2, rs, cs], i2v)
            for r in range(TROW):
                def body(j, carry):
                    c0 = j * (4 * L)
                    for u in range(4):
                        col = pl.ds(c0 + u * L, L)
                        i1 = i1v[r, col]
                        i2 = i2v[r, col]
                        idx = jnp.maximum(jnp.minimum(i1, 255), 0)
                        rcp = plsc.load_gather(rcp_v, [idx])
                        o1v[r, col] = i1.astype(jnp.float32)
                        o2v[r, col] = i2.astype(jnp.float32) * rcp
                    return carry
                lax.fori_loop(0, HALF // (4 * L), body, 0)
            pltpu.sync_copy(o1v, out_hbm.at[1, rs, cs])
            pltpu.sync_copy(o2v, out_hbm.at[2, rs, cs])

        def cast_half(u):
            rs = pl.ds((u // 2) * TROW, TROW)
            cs = pl.ds((u % 2) * HALF, HALF)
            pltpu.sync_copy(x_hbm.at[0, rs, cs], i1v)
            for r in range(TROW):
                def body(j, carry):
                    c0 = j * (4 * L)
                    for uu in range(4):
                        col = pl.ds(c0 + uu * L, L)
                        o1v[r, col] = i1v[r, col].astype(jnp.float32)
                    return carry
                lax.fori_loop(0, HALF // (4 * L), body, 0)
            pltpu.sync_copy(o1v, out_hbm.at[0, rs, cs])

        @pl.when(w < NTROW)
        def _():
            joint_half(w, 0)
            joint_half(w, 1)

        base = jnp.where(w < NTROW, w - 10, 15 + 5 * (w - NTROW))
        n = jnp.where(w < 10, 0, jnp.where(w < NTROW, 1, 5))
        for k in range(5):
            @pl.when(k < n)
            def _():
                cast_half(base + k)

    ot = sc_kernel(xt, norm_factors)
    return jnp.transpose(ot, (2, 1, 0))


# speedup vs baseline: 130.1887x; 1.4696x over previous
import functools

import jax
import jax.numpy as jnp
from jax import lax
from jax.experimental import pallas as pl
from jax.experimental.pallas import tpu as pltpu
from jax.experimental.pallas import tpu_sc as plsc

B = 4096
T = 200
NPLANE = 3
ROWS = T
COLS = B
TROW = 8
NTROW = ROWS // TROW
HALF = COLS // 2
L = 16


def kernel(observations, norm_factors):
    xt = jnp.transpose(observations, (2, 1, 0))

    mesh = plsc.VectorSubcoreMesh(core_axis_name="c", subcore_axis_name="s")

    @functools.partial(
        pl.kernel,
        mesh=mesh,
        out_type=jax.ShapeDtypeStruct((NPLANE, ROWS, COLS), jnp.float32),
        compiler_params=pltpu.CompilerParams(needs_layout_passes=False),
        scratch_types=[
            pltpu.VMEM((256,), jnp.float32),
            pltpu.VMEM((256,), jnp.float32),
            pltpu.VMEM((TROW, HALF), jnp.int32),
            pltpu.VMEM((TROW, HALF), jnp.int32),
            pltpu.VMEM((TROW, HALF), jnp.float32),
            pltpu.VMEM((TROW, HALF), jnp.float32),
        ],
    )
    def sc_kernel(x_hbm, nf_hbm, out_hbm, nf_v, rcp_v, i1v, i2v, o1v, o2v):
        w = lax.axis_index("s") * 2 + lax.axis_index("c")

        pltpu.sync_copy(nf_hbm, nf_v)
        for i in range(256 // L):
            rcp_v[pl.ds(i * L, L)] = 1.0 / nf_v[pl.ds(i * L, L)]

        def joint_half(row, h):
            rs = pl.ds(row * TROW, TROW)
            cs = pl.ds(h * HALF, HALF)
            pltpu.sync_copy(x_hbm.at[1, rs, cs], i1v)
            pltpu.sync_copy(x_hbm.at[2, rs, cs], i2v)
            for r in range(TROW):
                @plsc.parallel_loop(0, HALF // L, unroll=8)
                def _(j):
                    col = pl.ds(j * L, L)
                    i1 = i1v[r, col]
                    i2 = i2v[r, col]
                    idx = jnp.maximum(jnp.minimum(i1, 255), 0)
                    rcp = plsc.load_gather(rcp_v, [idx])
                    o1v[r, col] = i1.astype(jnp.float32)
                    o2v[r, col] = i2.astype(jnp.float32) * rcp
            pltpu.sync_copy(o1v, out_hbm.at[1, rs, cs])
            pltpu.sync_copy(o2v, out_hbm.at[2, rs, cs])

        def cast_half(u):
            rs = pl.ds((u // 2) * TROW, TROW)
            cs = pl.ds((u % 2) * HALF, HALF)
            pltpu.sync_copy(x_hbm.at[0, rs, cs], i1v)
            for r in range(TROW):
                @plsc.parallel_loop(0, HALF // L, unroll=8)
                def _(j):
                    col = pl.ds(j * L, L)
                    o1v[r, col] = i1v[r, col].astype(jnp.float32)
            pltpu.sync_copy(o1v, out_hbm.at[0, rs, cs])

        @pl.when(w < NTROW)
        def _():
            joint_half(w, 0)
            joint_half(w, 1)

        base = jnp.where(w < NTROW, w - 10, 15 + 5 * (w - NTROW))
        n = jnp.where(w < 10, 0, jnp.where(w < NTROW, 1, 5))
        for k in range(5):
            @pl.when(k < n)
            def _():
                cast_half(base + k)

    ot = sc_kernel(xt, norm_factors)
    return jnp.transpose(ot, (2, 1, 0))


# speedup vs baseline: 164.6551x; 1.2647x over previous
import functools

import jax
import jax.numpy as jnp
from jax import lax
from jax.experimental import pallas as pl
from jax.experimental.pallas import tpu as pltpu
from jax.experimental.pallas import tpu_sc as plsc

B = 4096
T = 200
NPLANE = 3
ROWS = T
COLS = B
TROW = 8
QW = COLS // 4
L = 16


def kernel(observations, norm_factors):
    xt = jnp.transpose(observations, (2, 1, 0))

    mesh = plsc.VectorSubcoreMesh(core_axis_name="c", subcore_axis_name="s")

    @functools.partial(
        pl.kernel,
        mesh=mesh,
        out_type=jax.ShapeDtypeStruct((NPLANE, ROWS, COLS), jnp.float32),
        compiler_params=pltpu.CompilerParams(needs_layout_passes=False),
        scratch_types=[
            pltpu.VMEM((256,), jnp.float32),
            pltpu.VMEM((256,), jnp.float32),
            pltpu.VMEM((TROW, QW), jnp.int32),
            pltpu.VMEM((TROW, QW), jnp.int32),
            pltpu.VMEM((TROW, QW), jnp.int32),
            pltpu.VMEM((TROW, QW), jnp.int32),
            pltpu.VMEM((TROW, QW), jnp.float32),
            pltpu.VMEM((TROW, QW), jnp.float32),
            pltpu.VMEM((TROW, QW), jnp.float32),
            pltpu.VMEM((TROW, QW), jnp.float32),
            pltpu.SemaphoreType.DMA,
            pltpu.SemaphoreType.DMA,
            pltpu.SemaphoreType.DMA,
            pltpu.SemaphoreType.DMA,
        ],
    )
    def sc_kernel(x_hbm, nf_hbm, out_hbm, nf_v, rcp_v,
                  i1a, i1b, i2a, i2b, o1a, o1b, o2a, o2b,
                  in_sa, in_sb, out_sa, out_sb):
        w = lax.axis_index("s") * 2 + lax.axis_index("c")

        nj = jnp.where(w < 4, 4, 3)
        base_j = 3 * w + jnp.minimum(w, 4)
        nc = jnp.where(w >= 28, 4, 3)
        base_c = 3 * w + jnp.maximum(w - 28, 0)

        i1 = (i1a, i1b)
        i2 = (i2a, i2b)
        o1 = (o1a, o1b)
        o2 = (o2a, o2b)
        in_s = (in_sa, in_sb)
        out_s = (out_sa, out_sb)

        def slices(u):
            return pl.ds((u // 4) * TROW, TROW), pl.ds((u % 4) * QW, QW)

        def in_joint(u, s):
            rs, cs = slices(u)
            return (pltpu.make_async_copy(x_hbm.at[1, rs, cs], i1[s], in_s[s]),
                    pltpu.make_async_copy(x_hbm.at[2, rs, cs], i2[s], in_s[s]))

        def out_joint(u, s):
            rs, cs = slices(u)
            return (pltpu.make_async_copy(o1[s], out_hbm.at[1, rs, cs], out_s[s]),
                    pltpu.make_async_copy(o2[s], out_hbm.at[2, rs, cs], out_s[s]))

        def in_cast(u, s):
            rs, cs = slices(u)
            return (pltpu.make_async_copy(x_hbm.at[0, rs, cs], i1[s], in_s[s]),)

        def out_cast(u, s):
            rs, cs = slices(u)
            return (pltpu.make_async_copy(o1[s], out_hbm.at[0, rs, cs], out_s[s]),)

        def start(copies):
            for c in copies:
                c.start()

        def wait(copies):
            for c in copies:
                c.wait()

        def compute_joint(s):
            i1s, i2s, o1s, o2s = i1[s], i2[s], o1[s], o2[s]

            def row_body(r, carry):
                @plsc.parallel_loop(0, QW // L, unroll=8)
                def _(j):
                    col = pl.ds(j * L, L)
                    a = i1s[r, col]
                    v = i2s[r, col]
                    idx = jnp.maximum(jnp.minimum(a, 255), 0)
                    rcp = plsc.load_gather(rcp_v, [idx])
                    o1s[r, col] = a.astype(jnp.float32)
                    o2s[r, col] = v.astype(jnp.float32) * rcp
                return carry

            lax.fori_loop(0, TROW, row_body, 0)

        def compute_cast(s):
            i1s, o1s = i1[s], o1[s]

            def row_body(r, carry):
                @plsc.parallel_loop(0, QW // L, unroll=8)
                def _(j):
                    col = pl.ds(j * L, L)
                    o1s[r, col] = i1s[r, col].astype(jnp.float32)
                return carry

            lax.fori_loop(0, TROW, row_body, 0)

        start(in_joint(base_j + 0, 0))
        start(in_joint(base_j + 1, 1))

        pltpu.sync_copy(nf_hbm, nf_v)
        for i in range(256 // L):
            rcp_v[pl.ds(i * L, L)] = 1.0 / nf_v[pl.ds(i * L, L)]

        for q in range(4):
            s = q & 1
            u = base_j + q

            def iter_body(u=u, s=s, q=q):
                wait(in_joint(u, s))
                if q >= 2:
                    wait(out_joint(base_j + (q - 2), s))
                compute_joint(s)
                start(out_joint(u, s))
                if q + 2 < 4:
                    @pl.when(q + 2 < nj)
                    def _():
                        start(in_joint(base_j + (q + 2), s))

            if q < 3:
                iter_body()
            else:
                pl.when(q < nj)(iter_body)

        start(in_cast(base_c + 0, 0))
        start(in_cast(base_c + 1, 1))

        @pl.when(nj == 3)
        def _():
            wait(out_joint(base_j + 1, 1))
        wait(out_joint(base_j + 2, 0))
        @pl.when(nj == 4)
        def _():
            wait(out_joint(base_j + 3, 1))

        for q in range(4):
            s = q & 1
            u = base_c + q

            def iter_body(u=u, s=s, q=q):
                wait(in_cast(u, s))
                if q >= 2:
                    wait(out_cast(base_c + (q - 2), s))
                compute_cast(s)
                start(out_cast(u, s))
                if q + 2 < 4:
                    @pl.when(q + 2 < nc)
                    def _():
                        start(in_cast(base_c + (q + 2), s))

            if q < 3:
                iter_body()
            else:
                pl.when(q < nc)(iter_body)

        @pl.when(nc == 3)
        def _():
            wait(out_cast(base_c + 1, 1))
        wait(out_cast(base_c + 2, 0))
        @pl.when(nc == 4)
        def _():
            wait(out_cast(base_c + 3, 1))

    ot = sc_kernel(xt, norm_factors)
    return jnp.transpose(ot, (2, 1, 0))
